# Initial kernel scaffold; baseline (speedup 1.0000x reference)
import functools
import math

import jax
import jax.numpy as jnp
import numpy as np
from jax import lax
from jax.experimental import pallas as pl
from jax.experimental.pallas import tpu as pltpu
from jax.experimental.pallas import tpu_sc as plsc

_N_NODES = 10000
_E = 160000
_NF = 128
_NRBF = 20
_RCUT = 5.0

_B = 640
_NT = 16
_CHUNK = 80
_PER_TILE = _E // _NT
_NCHUNK = _PER_TILE // _CHUNK
_ROWS_T = _N_NODES // _NT
_ZROWS = 125


def _expansion_mats():
    R = np.zeros((_NF, 3 * _NF), np.float32)
    for f in range(_NF):
        R[f, 3 * f:3 * f + 3] = 1.0
    Rr = np.zeros((8, 3 * _NF), np.float32)
    for f in range(_NF):
        for c in range(3):
            Rr[c, 3 * f + c] = 1.0
    return R, Rr


_R_NP, _RR_NP = _expansion_mats()


def _dot(a, b):
    return jnp.dot(a, b, preferred_element_type=jnp.float32,
                   precision=lax.Precision.HIGHEST)


def _tc_body(sj, vjr, rpad, W1, b1, W2, b2, Wrp, br, R, Rr, out):
    r = rpad[...]
    sq = jnp.sum(r * r, axis=1, keepdims=True)
    rn = jnp.sqrt(sq)
    inv = 1.0 / (rn + 1e-8)
    rhat = r * inv
    k = lax.broadcasted_iota(jnp.float32, (_B, _NF), 1) + 1.0
    rbf = jnp.where(k <= _NRBF,
                    jnp.sin(k * (math.pi / _RCUT) * rn) * inv,
                    0.0)
    fcut = 0.5 * (jnp.cos((math.pi / _RCUT) * rn) + 1.0)
    fcut = jnp.where(rn > _RCUT, 0.0, fcut)
    ws = (_dot(rbf, Wrp[...]) + br[...]) * fcut
    h = _dot(sj[...], W1[...]) + b1[...]
    h = h * jax.nn.sigmoid(h)
    phi = _dot(h, W2[...]) + b2[...]
    phiw = phi * ws
    s1 = phiw[:, :_NF]
    s2 = phiw[:, _NF:2 * _NF]
    s3 = phiw[:, 2 * _NF:]
    cv = vjr[...] * _dot(s1, R[...]) + _dot(rhat, Rr[...]) * _dot(s3, R[...])
    out[...] = jnp.concatenate([cv, s2], axis=1)


_tc_call = pl.pallas_call(
    _tc_body,
    grid=(_E // _B,),
    in_specs=[
        pl.BlockSpec((_B, _NF), lambda i: (i, 0)),
        pl.BlockSpec((_B, 3 * _NF), lambda i: (i, 0)),
        pl.BlockSpec((_B, 8), lambda i: (i, 0)),
        pl.BlockSpec((_NF, _NF), lambda i: (0, 0)),
        pl.BlockSpec((1, _NF), lambda i: (0, 0)),
        pl.BlockSpec((_NF, 3 * _NF), lambda i: (0, 0)),
        pl.BlockSpec((1, 3 * _NF), lambda i: (0, 0)),
        pl.BlockSpec((_NF, 3 * _NF), lambda i: (0, 0)),
        pl.BlockSpec((1, 3 * _NF), lambda i: (0, 0)),
        pl.BlockSpec((_NF, 3 * _NF), lambda i: (0, 0)),
        pl.BlockSpec((8, 3 * _NF), lambda i: (0, 0)),
    ],
    out_specs=pl.BlockSpec((_B, 512), lambda i: (i, 0)),
    out_shape=jax.ShapeDtypeStruct((_E, 512), jnp.float32),
)


def _sc_body(contrib, dsts, out, buf, idx, zbuf, acc):
    c = lax.axis_index("c")
    s = lax.axis_index("s")

    def zrow(i, carry):
        def zcol(j, carry2):
            zbuf[i, pl.ds(j * 16, 16)] = jnp.zeros((16,), jnp.float32)
            return carry2
        return lax.fori_loop(0, 8, zcol, carry)
    lax.fori_loop(0, _ZROWS, zrow, 0)

    row0 = s * _ROWS_T
    base = s * _PER_TILE
    for rnd in range(2):
        g0 = (c * 2 + rnd) * 128
        for i in range(_ROWS_T // _ZROWS):
            pltpu.sync_copy(zbuf, acc.at[pl.ds(row0 + i * _ZROWS, _ZROWS)])
        plsc.subcore_barrier()

        def chunk(j, carry):
            e0 = base + j * _CHUNK
            pltpu.sync_copy(dsts.at[pl.ds(e0, _CHUNK)], idx)
            pltpu.sync_copy(contrib.at[pl.ds(e0, _CHUNK), pl.ds(g0, 128)], buf)
            pltpu.sync_copy(buf, acc.at[idx], add=True)
            return carry
        lax.fori_loop(0, _NCHUNK, chunk, 0)
        plsc.subcore_barrier()
        for i in range(_ROWS_T // _ZROWS):
            rr = row0 + i * _ZROWS
            pltpu.sync_copy(acc.at[pl.ds(rr, _ZROWS)],
                            out.at[pl.ds(rr, _ZROWS), pl.ds(g0, 128)])


_sc_call = functools.partial(
    pl.kernel,
    out_type=jax.ShapeDtypeStruct((_N_NODES, 512), jnp.float32),
    mesh=plsc.VectorSubcoreMesh(core_axis_name="c", subcore_axis_name="s"),
    scratch_types=[
        pltpu.VMEM((_CHUNK, 128), jnp.float32),
        pltpu.VMEM((_CHUNK,), jnp.int32),
        pltpu.VMEM((_ZROWS, 128), jnp.float32),
        pltpu.VMEM_SHARED((_N_NODES, 128), jnp.float32),
    ],
)(_sc_body)


def kernel(vj, sj, rij_vec, eij, W1, b1, W2, b2, Wr, br):
    vjr = vj.reshape(_E, 3 * _NF)
    rpad = jnp.pad(rij_vec, ((0, 0), (0, 5)))
    Wrp = jnp.pad(Wr, ((0, _NF - _NRBF), (0, 0)))
    contrib = _tc_call(sj, vjr, rpad, W1, b1.reshape(1, _NF), W2,
                       b2.reshape(1, 3 * _NF), Wrp, br.reshape(1, 3 * _NF),
                       jnp.asarray(_R_NP), jnp.asarray(_RR_NP))
    dst = eij[1]
    out = _sc_call(contrib, dst)
    d_vim = out[:, :3 * _NF].reshape(_N_NODES, _NF, 3)
    d_sim = out[:, 3 * _NF:]
    return (d_vim, d_sim)


# speedup vs baseline: 6.1348x; 6.1348x over previous
import functools
import math

import jax
import jax.numpy as jnp
import numpy as np
from jax import lax
from jax.experimental import pallas as pl
from jax.experimental.pallas import tpu as pltpu
from jax.experimental.pallas import tpu_sc as plsc

_N_NODES = 10000
_E = 160000
_NF = 128
_NRBF = 20
_RCUT = 5.0

_B = 640
_NT = 16
_CHUNK = 80
_PER_TILE = _E // _NT
_NCHUNK = _PER_TILE // _CHUNK
_ROWS_T = 624
_TAIL = _N_NODES - _NT * _ROWS_T
_ZROWS = 208


def _expansion_mats():
    R = np.zeros((_NF, 3 * _NF), np.float32)
    for f in range(_NF):
        R[f, 3 * f:3 * f + 3] = 1.0
    Rr = np.zeros((8, 3 * _NF), np.float32)
    for f in range(_NF):
        for c in range(3):
            Rr[c, 3 * f + c] = 1.0
    return R, Rr


_R_NP, _RR_NP = _expansion_mats()


def _dot(a, b):
    return jnp.dot(a, b, preferred_element_type=jnp.float32,
                   precision=lax.Precision.HIGHEST)


def _tc_body(sj, vjr, rpad, W1, b1, W2, b2, Wrp, br, R, Rr, out):
    r = rpad[...]
    sq = jnp.sum(r * r, axis=1, keepdims=True)
    rn = jnp.sqrt(sq)
    inv = 1.0 / (rn + 1e-8)
    rhat = r * inv
    k = lax.broadcasted_iota(jnp.int32, (_B, _NF), 1).astype(jnp.float32) + 1.0
    rbf = jnp.where(k <= _NRBF,
                    jnp.sin(k * (math.pi / _RCUT) * rn) * inv,
                    0.0)
    fcut = 0.5 * (jnp.cos((math.pi / _RCUT) * rn) + 1.0)
    fcut = jnp.where(rn > _RCUT, 0.0, fcut)
    ws = (_dot(rbf, Wrp[...]) + br[...]) * fcut
    h = _dot(sj[...], W1[...]) + b1[...]
    h = h * jax.nn.sigmoid(h)
    phi = _dot(h, W2[...]) + b2[...]
    phiw = phi * ws
    s1 = phiw[:, :_NF]
    s2 = phiw[:, _NF:2 * _NF]
    s3 = phiw[:, 2 * _NF:]
    cv = vjr[...] * _dot(s1, R[...]) + _dot(rhat, Rr[...]) * _dot(s3, R[...])
    out[...] = jnp.concatenate([cv, s2], axis=1)


_tc_call = pl.pallas_call(
    _tc_body,
    grid=(_E // _B,),
    in_specs=[
        pl.BlockSpec((_B, _NF), lambda i: (i, 0)),
        pl.BlockSpec((_B, 3 * _NF), lambda i: (i, 0)),
        pl.BlockSpec((_B, 8), lambda i: (i, 0)),
        pl.BlockSpec((_NF, _NF), lambda i: (0, 0)),
        pl.BlockSpec((1, _NF), lambda i: (0, 0)),
        pl.BlockSpec((_NF, 3 * _NF), lambda i: (0, 0)),
        pl.BlockSpec((1, 3 * _NF), lambda i: (0, 0)),
        pl.BlockSpec((_NF, 3 * _NF), lambda i: (0, 0)),
        pl.BlockSpec((1, 3 * _NF), lambda i: (0, 0)),
        pl.BlockSpec((_NF, 3 * _NF), lambda i: (0, 0)),
        pl.BlockSpec((8, 3 * _NF), lambda i: (0, 0)),
    ],
    out_specs=pl.BlockSpec((_B, 512), lambda i: (i, 0)),
    out_shape=jax.ShapeDtypeStruct((_E, 512), jnp.float32),
)


def _sc_body(contrib, dsts, out, buf, idx, zbuf, acc):
    c = lax.axis_index("c")
    s = lax.axis_index("s")

    def zrow(i, carry):
        def zcol(j, carry2):
            zbuf[i, pl.ds(j * 16, 16)] = jnp.zeros((16,), jnp.float32)
            return carry2
        return lax.fori_loop(0, 8, zcol, carry)
    lax.fori_loop(0, _ZROWS, zrow, 0)

    row0 = pl.multiple_of(s * _ROWS_T, 8)
    base = pl.multiple_of(s * _PER_TILE, 8)
    tail0 = _NT * _ROWS_T
    for rnd in range(2):
        g0 = pl.multiple_of((c * 2 + rnd) * 128, 128)
        for i in range(_ROWS_T // _ZROWS):
            rr = pl.multiple_of(row0 + i * _ZROWS, 8)
            pltpu.sync_copy(zbuf, acc.at[pl.ds(rr, _ZROWS)])
        @pl.when(s == _NT - 1)
        def _zero_tail():
            pltpu.sync_copy(zbuf.at[pl.ds(0, _TAIL)],
                            acc.at[pl.ds(tail0, _TAIL)])
        plsc.subcore_barrier()

        def chunk(j, carry):
            e0 = pl.multiple_of(base + j * _CHUNK, 8)
            pltpu.sync_copy(dsts.at[pl.ds(e0, _CHUNK)], idx)
            pltpu.sync_copy(contrib.at[pl.ds(e0, _CHUNK), pl.ds(g0, 128)], buf)
            pltpu.sync_copy(buf, acc.at[idx], add=True)
            return carry
        lax.fori_loop(0, _NCHUNK, chunk, 0)
        plsc.subcore_barrier()
        for i in range(_ROWS_T // _ZROWS):
            rr = pl.multiple_of(row0 + i * _ZROWS, 8)
            pltpu.sync_copy(acc.at[pl.ds(rr, _ZROWS)],
                            out.at[pl.ds(rr, _ZROWS), pl.ds(g0, 128)])
        @pl.when(s == _NT - 1)
        def _write_tail():
            pltpu.sync_copy(acc.at[pl.ds(tail0, _TAIL)],
                            out.at[pl.ds(tail0, _TAIL), pl.ds(g0, 128)])


@functools.cache
def _sc_call():
    return pl.kernel(
        _sc_body,
        out_type=jax.ShapeDtypeStruct((_N_NODES, 512), jnp.float32),
        mesh=plsc.VectorSubcoreMesh(core_axis_name="c", subcore_axis_name="s"),
        scratch_types=[
            pltpu.VMEM((_CHUNK, 128), jnp.float32),
            pltpu.VMEM((_CHUNK,), jnp.int32),
            pltpu.VMEM((_ZROWS, 128), jnp.float32),
            pltpu.VMEM_SHARED((_N_NODES, 128), jnp.float32),
        ],
    )


def kernel(vj, sj, rij_vec, eij, W1, b1, W2, b2, Wr, br):
    vjr = vj.reshape(_E, 3 * _NF)
    rpad = jnp.pad(rij_vec, ((0, 0), (0, 5)))
    Wrp = jnp.pad(Wr, ((0, _NF - _NRBF), (0, 0)))
    contrib = _tc_call(sj, vjr, rpad, W1, b1.reshape(1, _NF), W2,
                       b2.reshape(1, 3 * _NF), Wrp, br.reshape(1, 3 * _NF),
                       jnp.asarray(_R_NP), jnp.asarray(_RR_NP))
    dst = eij[1]
    out = _sc_call()(contrib, dst)
    d_vim = out[:, :3 * _NF].reshape(_N_NODES, _NF, 3)
    d_sim = out[:, 3 * _NF:]
    return (d_vim, d_sim)


# speedup vs baseline: 9.1615x; 1.4934x over previous
import functools
import math

import jax
import jax.numpy as jnp
import numpy as np
from jax import lax
from jax.experimental import pallas as pl
from jax.experimental.pallas import tpu as pltpu
from jax.experimental.pallas import tpu_sc as plsc

_N_NODES = 10000
_E = 160000
_NF = 128
_NRBF = 20
_RCUT = 5.0

_B = 640
_NT = 16
_CHUNK = 80
_PER_TILE = _E // _NT
_NCHUNK = _PER_TILE // _CHUNK
_ROWS_T = 624
_TAIL = _N_NODES - _NT * _ROWS_T
_ZROWS = 208


def _expansion_mats():
    R = np.zeros((_NF, 3 * _NF), np.float32)
    for f in range(_NF):
        R[f, 3 * f:3 * f + 3] = 1.0
    Rr = np.zeros((8, 3 * _NF), np.float32)
    for f in range(_NF):
        for c in range(3):
            Rr[c, 3 * f + c] = 1.0
    return R, Rr


_R_NP, _RR_NP = _expansion_mats()


def _dot(a, b):
    return jnp.dot(a, b, preferred_element_type=jnp.float32)


def _tc_body(sj, vjr, rpad, W1, b1, W2, b2, Wrp, br, R, Rr,
             out0, out1, out2, out3):
    r = rpad[...]
    sq = jnp.sum(r * r, axis=1, keepdims=True)
    rn = jnp.sqrt(sq)
    inv = 1.0 / (rn + 1e-8)
    rhat = r * inv
    k = lax.broadcasted_iota(jnp.int32, (_B, _NF), 1).astype(jnp.float32) + 1.0
    rbf = jnp.where(k <= _NRBF,
                    jnp.sin(k * (math.pi / _RCUT) * rn) * inv,
                    0.0)
    fcut = 0.5 * (jnp.cos((math.pi / _RCUT) * rn) + 1.0)
    fcut = jnp.where(rn > _RCUT, 0.0, fcut)
    ws = (_dot(rbf, Wrp[...]) + br[...]) * fcut
    h = _dot(sj[...], W1[...]) + b1[...]
    h = h * jax.nn.sigmoid(h)
    phi = _dot(h, W2[...]) + b2[...]
    phiw = phi * ws
    s1 = phiw[:, :_NF]
    s2 = phiw[:, _NF:2 * _NF]
    s3 = phiw[:, 2 * _NF:]
    cv = vjr[...] * _dot(s1, R[...]) + _dot(rhat, Rr[...]) * _dot(s3, R[...])
    out0[...] = cv[:, :_NF]
    out1[...] = cv[:, _NF:2 * _NF]
    out2[...] = cv[:, 2 * _NF:]
    out3[...] = s2


_edge_spec = pl.BlockSpec((_B, _NF), lambda i: (i, 0))
_tc_call = pl.pallas_call(
    _tc_body,
    grid=(_E // _B,),
    in_specs=[
        pl.BlockSpec((_B, _NF), lambda i: (i, 0)),
        pl.BlockSpec((_B, 3 * _NF), lambda i: (i, 0)),
        pl.BlockSpec((_B, 8), lambda i: (i, 0)),
        pl.BlockSpec((_NF, _NF), lambda i: (0, 0)),
        pl.BlockSpec((1, _NF), lambda i: (0, 0)),
        pl.BlockSpec((_NF, 3 * _NF), lambda i: (0, 0)),
        pl.BlockSpec((1, 3 * _NF), lambda i: (0, 0)),
        pl.BlockSpec((_NF, 3 * _NF), lambda i: (0, 0)),
        pl.BlockSpec((1, 3 * _NF), lambda i: (0, 0)),
        pl.BlockSpec((_NF, 3 * _NF), lambda i: (0, 0)),
        pl.BlockSpec((8, 3 * _NF), lambda i: (0, 0)),
    ],
    out_specs=[_edge_spec, _edge_spec, _edge_spec, _edge_spec],
    out_shape=[jax.ShapeDtypeStruct((_E, _NF), jnp.float32)] * 4,
)


def _sc_body(cref0, cref1, cref2, cref3, dsts,
             out0, out1, out2, out3,
             buf0, buf1, idx0, idx1, zbuf, acc, sem):
    c = lax.axis_index("c")
    s = lax.axis_index("s")
    crefs = [cref0, cref1, cref2, cref3]
    outs = [out0, out1, out2, out3]
    bufs = [buf0, buf1]
    idxs = [idx0, idx1]

    def zrow(i, carry):
        def zcol(j, carry2):
            zbuf[i, pl.ds(j * 16, 16)] = jnp.zeros((16,), jnp.float32)
            return carry2
        return lax.fori_loop(0, 8, zcol, carry)
    lax.fori_loop(0, _ZROWS, zrow, 0)

    row0 = pl.multiple_of(s * _ROWS_T, 8)
    base = pl.multiple_of(s * _PER_TILE, 8)
    tail0 = _NT * _ROWS_T

    for gi in range(4):
        @pl.when(c == gi // 2)
        def _round(gi=gi):
            cref = crefs[gi]
            outg = outs[gi]

            for i in range(_ROWS_T // _ZROWS):
                rr = pl.multiple_of(row0 + i * _ZROWS, 8)
                pltpu.sync_copy(zbuf, acc.at[pl.ds(rr, _ZROWS)])

            @pl.when(s == _NT - 1)
            def _zero_tail():
                pltpu.sync_copy(zbuf.at[pl.ds(0, _TAIL)],
                                acc.at[pl.ds(tail0, _TAIL)])
            plsc.subcore_barrier()

            def _start(j, slot):
                e0 = pl.multiple_of(base + j * _CHUNK, 8)
                pltpu.async_copy(dsts.at[pl.ds(e0, _CHUNK)], idxs[slot], sem)
                pltpu.async_copy(cref.at[pl.ds(e0, _CHUNK)], bufs[slot], sem)

            def _drain(j, slot):
                e0 = pl.multiple_of(base + j * _CHUNK, 8)
                pltpu.make_async_copy(dsts.at[pl.ds(e0, _CHUNK)],
                                      idxs[slot], sem).wait()
                pltpu.make_async_copy(cref.at[pl.ds(e0, _CHUNK)],
                                      bufs[slot], sem).wait()

            _start(0, 0)

            def outer(j2, carry):
                for b in range(2):
                    j = j2 + b

                    @pl.when(j < _NCHUNK)
                    def _step(j=j, b=b):
                        _drain(j, b)

                        @pl.when(j + 1 < _NCHUNK)
                        def _prefetch():
                            _start(j + 1, 1 - b)
                        pltpu.sync_copy(bufs[b], acc.at[idxs[b]], add=True)
                return carry
            lax.fori_loop(0, (_NCHUNK + 1) // 2, lambda t, cr: outer(t * 2, cr),
                          0)
            plsc.subcore_barrier()

            for i in range(_ROWS_T // _ZROWS):
                rr = pl.multiple_of(row0 + i * _ZROWS, 8)
                pltpu.sync_copy(acc.at[pl.ds(rr, _ZROWS)],
                                outg.at[pl.ds(rr, _ZROWS)])

            @pl.when(s == _NT - 1)
            def _write_tail():
                pltpu.sync_copy(acc.at[pl.ds(tail0, _TAIL)],
                                outg.at[pl.ds(tail0, _TAIL)])


@functools.cache
def _sc_call():
    return pl.kernel(
        _sc_body,
        out_type=[jax.ShapeDtypeStruct((_N_NODES, _NF), jnp.float32)] * 4,
        mesh=plsc.VectorSubcoreMesh(core_axis_name="c", subcore_axis_name="s"),
        scratch_types=[
            pltpu.VMEM((_CHUNK, _NF), jnp.float32),
            pltpu.VMEM((_CHUNK, _NF), jnp.float32),
            pltpu.VMEM((_CHUNK,), jnp.int32),
            pltpu.VMEM((_CHUNK,), jnp.int32),
            pltpu.VMEM((_ZROWS, _NF), jnp.float32),
            pltpu.VMEM_SHARED((_N_NODES, _NF), jnp.float32),
            pltpu.SemaphoreType.DMA,
        ],
    )


def kernel(vj, sj, rij_vec, eij, W1, b1, W2, b2, Wr, br):
    vjr = vj.reshape(_E, 3 * _NF)
    rpad = jnp.pad(rij_vec, ((0, 0), (0, 5)))
    Wrp = jnp.pad(Wr, ((0, _NF - _NRBF), (0, 0)))
    c0, c1, c2, c3 = _tc_call(sj, vjr, rpad, W1, b1.reshape(1, _NF), W2,
                              b2.reshape(1, 3 * _NF), Wrp,
                              br.reshape(1, 3 * _NF),
                              jnp.asarray(_R_NP), jnp.asarray(_RR_NP))
    dst = eij[1]
    o0, o1, o2, o3 = _sc_call()(c0, c1, c2, c3, dst)
    d_vim = jnp.concatenate([o0, o1, o2], axis=1).reshape(_N_NODES, _NF, 3)
    d_sim = o3
    return (d_vim, d_sim)


# speedup vs baseline: 12.0819x; 1.3188x over previous
import functools
import math

import jax
import jax.numpy as jnp
import numpy as np
from jax import lax
from jax.experimental import pallas as pl
from jax.experimental.pallas import tpu as pltpu
from jax.experimental.pallas import tpu_sc as plsc

_N_NODES = 10000
_E = 160000
_NF = 128
_NRBF = 20
_RCUT = 5.0

_B = 640
_NT = 16
_CHUNK = 80
_PER_TILE = _E // _NT
_NCHUNK = _PER_TILE // _CHUNK
_ROWS_T = 624
_TAIL = _N_NODES - _NT * _ROWS_T
_ZROWS = 208


def _expansion_mats():
    R = np.zeros((_NF, 3 * _NF), np.float32)
    for f in range(_NF):
        R[f, 3 * f:3 * f + 3] = 1.0
    Rr = np.zeros((8, 3 * _NF), np.float32)
    for f in range(_NF):
        for c in range(3):
            Rr[c, 3 * f + c] = 1.0
    return R, Rr


_R_NP, _RR_NP = _expansion_mats()


def _dot(a, b):
    return jnp.dot(a, b, preferred_element_type=jnp.float32)


def _tc_body(sj, vjr, rpadT, W1, b1, W2, b2, Wrpa, R, Rr,
             out0, out1, out2, out3):
    rT = rpadT[...]
    sqT = jnp.sum(rT * rT, axis=0, keepdims=True)
    rnT = jnp.sqrt(sqT)
    invT = 1.0 / (rnT + 1e-8)
    fcutT = jnp.where(rnT > _RCUT, 0.0,
                      0.5 * (jnp.cos((math.pi / _RCUT) * rnT) + 1.0))
    kcol = (lax.broadcasted_iota(jnp.int32, (32, 1), 0) + 1
            ).astype(jnp.float32)
    args = kcol * ((math.pi / _RCUT) * rnT)
    rbfT = jnp.sin(args) * (invT * fcutT)
    rowid = lax.broadcasted_iota(jnp.int32, (32, _B), 0)
    rbfT = jnp.where(rowid == _NRBF, fcutT, rbfT)
    rbf = rbfT.T
    ws = _dot(rbf, Wrpa[...])
    rhat = (rT * invT).T
    h = _dot(sj[...], W1[...]) + b1[...]
    h = h * jax.nn.sigmoid(h)
    phi = _dot(h, W2[...]) + b2[...]
    phiw = phi * ws
    s1 = phiw[:, :_NF]
    s2 = phiw[:, _NF:2 * _NF]
    s3 = phiw[:, 2 * _NF:]
    cv = vjr[...] * _dot(s1, R[...]) + _dot(rhat, Rr[...]) * _dot(s3, R[...])
    out0[...] = cv[:, :_NF].reshape(_B * _NF)
    out1[...] = cv[:, _NF:2 * _NF].reshape(_B * _NF)
    out2[...] = cv[:, 2 * _NF:].reshape(_B * _NF)
    out3[...] = s2.reshape(_B * _NF)


_edge_spec = pl.BlockSpec((_B * _NF,), lambda i: (i,))
_tc_call = pl.pallas_call(
    _tc_body,
    grid=(_E // _B,),
    in_specs=[
        pl.BlockSpec((_B, _NF), lambda i: (i, 0)),
        pl.BlockSpec((_B, 3 * _NF), lambda i: (i, 0)),
        pl.BlockSpec((8, _B), lambda i: (0, i)),
        pl.BlockSpec((_NF, _NF), lambda i: (0, 0)),
        pl.BlockSpec((1, _NF), lambda i: (0, 0)),
        pl.BlockSpec((_NF, 3 * _NF), lambda i: (0, 0)),
        pl.BlockSpec((1, 3 * _NF), lambda i: (0, 0)),
        pl.BlockSpec((32, 3 * _NF), lambda i: (0, 0)),
        pl.BlockSpec((_NF, 3 * _NF), lambda i: (0, 0)),
        pl.BlockSpec((8, 3 * _NF), lambda i: (0, 0)),
    ],
    out_specs=[_edge_spec, _edge_spec, _edge_spec, _edge_spec],
    out_shape=[jax.ShapeDtypeStruct((_E * _NF,), jnp.float32)] * 4,
)


def _sc_body(cref0, cref1, cref2, cref3, dsts,
             out0, out1, out2, out3,
             buf0, buf1, idx0, idx1, zbuf, acc, sem):
    c = lax.axis_index("c")
    s = lax.axis_index("s")
    crefs = [cref0, cref1, cref2, cref3]
    outs = [out0, out1, out2, out3]
    bufs = [buf0, buf1]
    idxs = [idx0, idx1]

    def zrow(i, carry):
        def zcol(j, carry2):
            zbuf[i, pl.ds(j * 16, 16)] = jnp.zeros((16,), jnp.float32)
            return carry2
        return lax.fori_loop(0, 8, zcol, carry)
    lax.fori_loop(0, _ZROWS, zrow, 0)

    row0 = pl.multiple_of(s * _ROWS_T, 8)
    base = pl.multiple_of(s * _PER_TILE, 8)
    tail0 = _NT * _ROWS_T

    for gi in range(4):
        @pl.when(c == gi // 2)
        def _round(gi=gi):
            cref = crefs[gi]
            outg = outs[gi]

            for i in range(_ROWS_T // _ZROWS):
                rr = pl.multiple_of(row0 + i * _ZROWS, 8)
                pltpu.sync_copy(zbuf, acc.at[pl.ds(rr, _ZROWS)])

            @pl.when(s == _NT - 1)
            def _zero_tail():
                pltpu.sync_copy(zbuf.at[pl.ds(0, _TAIL)],
                                acc.at[pl.ds(tail0, _TAIL)])
            plsc.subcore_barrier()

            def _start(j, slot):
                e0 = pl.multiple_of(base + j * _CHUNK, 8)
                pltpu.async_copy(dsts.at[pl.ds(e0, _CHUNK)], idxs[slot], sem)
                pltpu.async_copy(cref.at[pl.ds(e0, _CHUNK)], bufs[slot], sem)

            def _drain(j, slot):
                e0 = pl.multiple_of(base + j * _CHUNK, 8)
                pltpu.make_async_copy(dsts.at[pl.ds(e0, _CHUNK)],
                                      idxs[slot], sem).wait()
                pltpu.make_async_copy(cref.at[pl.ds(e0, _CHUNK)],
                                      bufs[slot], sem).wait()

            _start(0, 0)

            def outer(j2, carry):
                for b in range(2):
                    j = j2 + b

                    @pl.when(j < _NCHUNK)
                    def _step(j=j, b=b):
                        _drain(j, b)

                        @pl.when(j + 1 < _NCHUNK)
                        def _prefetch():
                            _start(j + 1, 1 - b)
                        pltpu.sync_copy(bufs[b], acc.at[idxs[b]], add=True)
                return carry
            lax.fori_loop(0, (_NCHUNK + 1) // 2, lambda t, cr: outer(t * 2, cr),
                          0)
            plsc.subcore_barrier()

            for i in range(_ROWS_T // _ZROWS):
                rr = pl.multiple_of(row0 + i * _ZROWS, 8)
                pltpu.sync_copy(acc.at[pl.ds(rr, _ZROWS)],
                                outg.at[pl.ds(rr, _ZROWS)])

            @pl.when(s == _NT - 1)
            def _write_tail():
                pltpu.sync_copy(acc.at[pl.ds(tail0, _TAIL)],
                                outg.at[pl.ds(tail0, _TAIL)])


@functools.cache
def _sc_call():
    return pl.kernel(
        _sc_body,
        out_type=[jax.ShapeDtypeStruct((_N_NODES, _NF), jnp.float32)] * 4,
        mesh=plsc.VectorSubcoreMesh(core_axis_name="c", subcore_axis_name="s"),
        scratch_types=[
            pltpu.VMEM((_CHUNK, _NF), jnp.float32),
            pltpu.VMEM((_CHUNK, _NF), jnp.float32),
            pltpu.VMEM((_CHUNK,), jnp.int32),
            pltpu.VMEM((_CHUNK,), jnp.int32),
            pltpu.VMEM((_ZROWS, _NF), jnp.float32),
            pltpu.VMEM_SHARED((_N_NODES, _NF), jnp.float32),
            pltpu.SemaphoreType.DMA,
        ],
    )


def kernel(vj, sj, rij_vec, eij, W1, b1, W2, b2, Wr, br):
    vjr = vj.reshape(_E, 3 * _NF)
    rpadT = jnp.pad(rij_vec.T, ((0, 5), (0, 0)))
    Wrpa = jnp.concatenate(
        [Wr, br.reshape(1, 3 * _NF),
         jnp.zeros((32 - _NRBF - 1, 3 * _NF), jnp.float32)], axis=0)
    c0, c1, c2, c3 = _tc_call(sj, vjr, rpadT, W1, b1.reshape(1, _NF), W2,
                              b2.reshape(1, 3 * _NF), Wrpa,
                              jnp.asarray(_R_NP), jnp.asarray(_RR_NP))
    dst = eij[1]
    o0, o1, o2, o3 = _sc_call()(c0.reshape(_E, _NF), c1.reshape(_E, _NF),
                                c2.reshape(_E, _NF), c3.reshape(_E, _NF), dst)
    d_vim = jnp.concatenate([o0, o1, o2], axis=1).reshape(_N_NODES, _NF, 3)
    d_sim = o3
    return (d_vim, d_sim)
